# Initial kernel scaffold; baseline (speedup 1.0000x reference)
import jax
import jax.numpy as jnp
from jax.experimental import pallas as pl


def kernel(users_feature, items_feature, graph_src, graph_dst, graph_values, Wq, bq, Wk, bk, Wv, bv, mask):
    raise NotImplementedError("write your pallas kernel here")


# speedup vs baseline: 10.5176x; 10.5176x over previous
import functools

import jax
import jax.numpy as jnp
from jax import lax
from jax.experimental import pallas as pl
from jax.experimental.pallas import tpu as pltpu
from jax.experimental.pallas import tpu_sc as plsc

_U, _I, _D = 65536, 4096, 64
_N = _U + _I
_E = 2097152
_KB = 8
_B = _KB * 128
_NTILES = 16
_ROWS_PER_TILE = _N // _NTILES
_ZR = _ROWS_PER_TILE // 4
_EROWS = _E // 128
_EROWS_PER_TILE = _EROWS // _NTILES
_BLOCKS = _EROWS_PER_TILE // _KB


def _sc_propagate(table, src2, dst2, val2):
    mesh = plsc.VectorSubcoreMesh(core_axis_name="c", subcore_axis_name="s")

    @functools.partial(
        pl.kernel,
        mesh=mesh,
        compiler_params=pltpu.CompilerParams(use_tc_tiling_on_sc=False),
        out_type=jax.ShapeDtypeStruct((_N, _D), jnp.float32),
        scratch_types=[
            pltpu.VMEM((_KB, 128), jnp.int32),
            pltpu.VMEM((_KB, 128), jnp.int32),
            pltpu.VMEM((_KB, 128), jnp.float32),
            pltpu.VMEM((_KB, 128, 16), jnp.float32),
            pltpu.VMEM((_ZR, 16), jnp.float32),
            pltpu.VMEM_SHARED((_N, 16), jnp.float32),
            pltpu.SemaphoreType.DMA,
        ],
    )
    def prop(table_ref, src_ref, dst_ref, val_ref, out_ref,
             idx_v, dst_v, val_v, rows_v, zero_v, acc, sem):
        c = lax.axis_index("c")
        s = lax.axis_index("s")

        def zbody(r, carry):
            zero_v[r] = jnp.zeros((16,), jnp.float32)
            return carry
        lax.fori_loop(0, _ZR, zbody, 0)

        row_base = s * _ROWS_PER_TILE
        erow_base = s * _EROWS_PER_TILE

        for p in range(2):
            k = c * 2 + p

            for z in range(4):
                pltpu.sync_copy(zero_v, acc.at[pl.ds(row_base + z * _ZR, _ZR), :])
            plsc.subcore_barrier()

            def block_body(b, carry):
                r0 = erow_base + b * _KB
                cp1 = pltpu.async_copy(src_ref.at[pl.ds(r0, _KB), :], idx_v, sem)
                cp2 = pltpu.async_copy(dst_ref.at[pl.ds(r0, _KB), :], dst_v, sem)
                cp3 = pltpu.async_copy(val_ref.at[pl.ds(r0, _KB), :], val_v, sem)
                cp1.wait()
                cp2.wait()
                cp3.wait()
                for rr in range(_KB):
                    for cc in range(8):
                        vsl = idx_v[rr, pl.ds(cc * 16, 16)]
                        idx_v[rr, pl.ds(cc * 16, 16)] = vsl * 4 + k
                gathers = [
                    pltpu.async_copy(table_ref.at[idx_v.at[j]], rows_v.at[j], sem)
                    for j in range(_KB)
                ]
                for g in gathers:
                    g.wait()
                for j in range(_KB):
                    def scale_body(g, carry2, j=j):
                        vv = val_v[j, pl.ds(g * 16, 16)]
                        base = g * 16
                        for t in range(16):
                            rows_v[j, base + t] = rows_v[j, base + t] * vv[t]
                        return carry2
                    lax.fori_loop(0, 8, scale_body, 0)
                for j in range(_KB):
                    pltpu.sync_copy(rows_v.at[j], acc.at[dst_v.at[j]], add=True)
                return carry

            lax.fori_loop(0, _BLOCKS, block_body, 0)
            plsc.subcore_barrier()
            pltpu.sync_copy(
                acc.at[pl.ds(row_base, _ROWS_PER_TILE), :],
                out_ref.at[pl.ds(row_base, _ROWS_PER_TILE), pl.ds(k * 16, 16)])
            plsc.subcore_barrier()

    return prop(table, src2, dst2, val2)


def _combine(feat, seg, br):
    n = feat.shape[0]

    def body(f_ref, g_ref, o_ref):
        f = f_ref[...]
        g = g_ref[...]
        ss = jnp.sum(g * g, axis=1, keepdims=True)
        denom = jnp.maximum(jnp.sqrt(ss), 2e-12)
        o_ref[...] = jnp.concatenate([f, g / denom], axis=1)

    return pl.pallas_call(
        body,
        grid=(n // br,),
        in_specs=[pl.BlockSpec((br, _D), lambda i: (i, 0)),
                  pl.BlockSpec((br, _D), lambda i: (i, 0))],
        out_specs=pl.BlockSpec((br, 2 * _D), lambda i: (i, 0)),
        out_shape=jax.ShapeDtypeStruct((n, 2 * _D), jnp.float32),
    )(feat, seg)


def _attention(items, Wq, bq, Wk, bk, Wv, bv, mask, bq_blk=512):
    bq2 = bq.reshape(1, _D)
    bk2 = bk.reshape(1, _D)
    bv2 = bv.reshape(1, _D)

    def body(q_ref, kv_ref, wq_ref, bq_ref, wk_ref, bk_ref, wv_ref, bv_ref,
             m_ref, o_ref):
        f32 = jnp.float32
        q = jnp.dot(q_ref[...], wq_ref[...], preferred_element_type=f32) + bq_ref[...]
        kv = kv_ref[...]
        kk = jnp.dot(kv, wk_ref[...], preferred_element_type=f32) + bk_ref[...]
        vv = jnp.dot(kv, wv_ref[...], preferred_element_type=f32) + bv_ref[...]
        s = lax.dot_general(q, kk, (((1,), (1,)), ((), ())),
                            preferred_element_type=f32)
        s = s * m_ref[...] * (1.0 / 8.0)
        m = jnp.max(s, axis=1, keepdims=True)
        p = jnp.exp(s - m)
        denom = jnp.sum(p, axis=1, keepdims=True)
        o = lax.dot_general(p, vv, (((1,), (0,)), ((), ())),
                            preferred_element_type=f32)
        o_ref[...] = o / denom

    return pl.pallas_call(
        body,
        grid=(_I // bq_blk,),
        in_specs=[
            pl.BlockSpec((bq_blk, _D), lambda i: (i, 0)),
            pl.BlockSpec((_I, _D), lambda i: (0, 0)),
            pl.BlockSpec((_D, _D), lambda i: (0, 0)),
            pl.BlockSpec((1, _D), lambda i: (0, 0)),
            pl.BlockSpec((_D, _D), lambda i: (0, 0)),
            pl.BlockSpec((1, _D), lambda i: (0, 0)),
            pl.BlockSpec((_D, _D), lambda i: (0, 0)),
            pl.BlockSpec((1, _D), lambda i: (0, 0)),
            pl.BlockSpec((bq_blk, _I), lambda i: (i, 0)),
        ],
        out_specs=pl.BlockSpec((bq_blk, _D), lambda i: (i, 0)),
        out_shape=jax.ShapeDtypeStruct((_I, _D), jnp.float32),
    )(items, items, Wq, bq2, Wk, bk2, Wv, bv2, mask)


def kernel(users_feature, items_feature, graph_src, graph_dst, graph_values,
           Wq, bq, Wk, bk, Wv, bv, mask):
    feats = jnp.concatenate([users_feature, items_feature], axis=0)
    table = feats.reshape(_N * 4, 16)
    src2 = graph_src.reshape(_EROWS, 128)
    dst2 = graph_dst.reshape(_EROWS, 128)
    val2 = graph_values.reshape(_EROWS, 128)

    seg = _sc_propagate(table, src2, dst2, val2)

    users_rep = _combine(users_feature, lax.slice(seg, (0, 0), (_U, _D)), 4096)
    items_rep = _combine(items_feature, lax.slice(seg, (_U, 0), (_N, _D)), 4096)
    attn = _attention(items_feature, Wq, bq, Wk, bk, Wv, bv, mask)
    return users_rep, items_rep, attn


# speedup vs baseline: 14.9198x; 1.4186x over previous
import functools

import jax
import jax.numpy as jnp
from jax import lax
from jax.experimental import pallas as pl
from jax.experimental.pallas import tpu as pltpu
from jax.experimental.pallas import tpu_sc as plsc

_U, _I, _D = 65536, 4096, 64
_N = _U + _I
_E = 2097152
_KB = 8
_B = _KB * 128
_NTILES = 16
_ROWS_PER_TILE = _N // _NTILES
_ZR = _ROWS_PER_TILE // 4
_GB = _E // _B
_BLOCKS = _GB // _NTILES
_PAIRS = _BLOCKS // 2


def _sc_propagate(table, packed, vals):
    mesh = plsc.VectorSubcoreMesh(core_axis_name="c", subcore_axis_name="s")

    @functools.partial(
        pl.kernel,
        mesh=mesh,
        compiler_params=pltpu.CompilerParams(use_tc_tiling_on_sc=False),
        out_type=jax.ShapeDtypeStruct((_N, _D), jnp.float32),
        scratch_types=[
            pltpu.VMEM((2, 2, _KB, 128), jnp.int32),
            pltpu.VMEM((2, _KB, 128), jnp.float32),
            pltpu.VMEM((2, _KB, 128), jnp.int32),
            pltpu.VMEM((2, _B, 16), jnp.float32),
            pltpu.VMEM((_ZR, 16), jnp.float32),
            pltpu.VMEM_SHARED((_N, 16), jnp.float32),
            pltpu.SemaphoreType.DMA,
            pltpu.SemaphoreType.DMA,
            pltpu.SemaphoreType.DMA,
        ],
    )
    def prop(table_ref, pk_ref, val_ref, out_ref,
             pk_v, val_v, gidx_v, rows_v, zero_v, acc, sem_a, sem_b, sem_sc):
        c = lax.axis_index("c")
        s = lax.axis_index("s")

        def zbody(r, carry):
            zero_v[r] = jnp.zeros((16,), jnp.float32)
            return carry
        lax.fori_loop(0, _ZR, zbody, 0)

        row_base = s * _ROWS_PER_TILE
        gblk_base = s * _BLOCKS

        def load_and_fire(b, buf, sem, k):
            pltpu.sync_copy(pk_ref.at[gblk_base + b], pk_v.at[buf])
            pltpu.sync_copy(val_ref.at[gblk_base + b], val_v.at[buf])
            for rr in range(_KB):
                for cc in range(8):
                    vsl = pk_v[buf, 0, rr, pl.ds(cc * 16, 16)]
                    gidx_v[buf, rr, pl.ds(cc * 16, 16)] = vsl * 4 + k
            for j in range(_KB):
                pltpu.async_copy(
                    table_ref.at[gidx_v.at[buf, j]],
                    rows_v.at[buf, pl.ds(j * 128, 128), :], sem)

        def drain(rows_slice, sem):
            pltpu.make_async_copy(
                table_ref.at[pl.ds(0, _B), :], rows_slice, sem).wait()

        def process(buf, sem):
            drain(rows_v.at[buf], sem)
            for j in range(_KB):
                def sb(g, carry, j=j, buf=buf):
                    vv = val_v[buf, j, pl.ds(g * 16, 16)]
                    base = j * 128 + g * 16
                    for t in range(16):
                        rows_v[buf, base + t] = rows_v[buf, base + t] * vv[t]
                    return carry
                lax.fori_loop(0, 8, sb, 0)
            for j in range(_KB):
                pltpu.async_copy(
                    rows_v.at[buf, pl.ds(j * 128, 128), :],
                    acc.at[pk_v.at[buf, 1, j]], sem_sc, add=True)
            drain(rows_v.at[buf], sem_sc)

        for p in range(2):
            k = c * 2 + p

            for z in range(4):
                pltpu.sync_copy(zero_v, acc.at[pl.ds(row_base + z * _ZR, _ZR), :])
            plsc.subcore_barrier()

            load_and_fire(0, 0, sem_a, k)

            def pair_body(i, carry, k=k):
                load_and_fire(2 * i + 1, 1, sem_b, k)
                process(0, sem_a)
                @pl.when(i < _PAIRS - 1)
                def _():
                    load_and_fire(2 * i + 2, 0, sem_a, k)
                process(1, sem_b)
                return carry
            lax.fori_loop(0, _PAIRS, pair_body, 0)

            plsc.subcore_barrier()
            pltpu.sync_copy(
                acc.at[pl.ds(row_base, _ROWS_PER_TILE), :],
                out_ref.at[pl.ds(row_base, _ROWS_PER_TILE), pl.ds(k * 16, 16)])
            plsc.subcore_barrier()

    return prop(table, packed, vals)


def _combine(feat, seg, br):
    n = feat.shape[0]

    def body(f_ref, g_ref, o_ref):
        f = f_ref[...]
        g = g_ref[...]
        ss = jnp.sum(g * g, axis=1, keepdims=True)
        denom = jnp.maximum(jnp.sqrt(ss), 2e-12)
        o_ref[...] = jnp.concatenate([f, g / denom], axis=1)

    return pl.pallas_call(
        body,
        grid=(n // br,),
        in_specs=[pl.BlockSpec((br, _D), lambda i: (i, 0)),
                  pl.BlockSpec((br, _D), lambda i: (i, 0))],
        out_specs=pl.BlockSpec((br, 2 * _D), lambda i: (i, 0)),
        out_shape=jax.ShapeDtypeStruct((n, 2 * _D), jnp.float32),
    )(feat, seg)


def _attention(items, Wq, bq, Wk, bk, Wv, bv, mask, bq_blk=512):
    bq2 = bq.reshape(1, _D)
    bk2 = bk.reshape(1, _D)
    bv2 = bv.reshape(1, _D)

    def body(q_ref, kv_ref, wq_ref, bq_ref, wk_ref, bk_ref, wv_ref, bv_ref,
             m_ref, o_ref):
        f32 = jnp.float32
        q = jnp.dot(q_ref[...], wq_ref[...], preferred_element_type=f32) + bq_ref[...]
        kv = kv_ref[...]
        kk = jnp.dot(kv, wk_ref[...], preferred_element_type=f32) + bk_ref[...]
        vv = jnp.dot(kv, wv_ref[...], preferred_element_type=f32) + bv_ref[...]
        s = lax.dot_general(q, kk, (((1,), (1,)), ((), ())),
                            preferred_element_type=f32)
        s = s * m_ref[...] * (1.0 / 8.0)
        m = jnp.max(s, axis=1, keepdims=True)
        p = jnp.exp(s - m)
        denom = jnp.sum(p, axis=1, keepdims=True)
        o = lax.dot_general(p, vv, (((1,), (0,)), ((), ())),
                            preferred_element_type=f32)
        o_ref[...] = o / denom

    return pl.pallas_call(
        body,
        grid=(_I // bq_blk,),
        in_specs=[
            pl.BlockSpec((bq_blk, _D), lambda i: (i, 0)),
            pl.BlockSpec((_I, _D), lambda i: (0, 0)),
            pl.BlockSpec((_D, _D), lambda i: (0, 0)),
            pl.BlockSpec((1, _D), lambda i: (0, 0)),
            pl.BlockSpec((_D, _D), lambda i: (0, 0)),
            pl.BlockSpec((1, _D), lambda i: (0, 0)),
            pl.BlockSpec((_D, _D), lambda i: (0, 0)),
            pl.BlockSpec((1, _D), lambda i: (0, 0)),
            pl.BlockSpec((bq_blk, _I), lambda i: (i, 0)),
        ],
        out_specs=pl.BlockSpec((bq_blk, _D), lambda i: (i, 0)),
        out_shape=jax.ShapeDtypeStruct((_I, _D), jnp.float32),
    )(items, items, Wq, bq2, Wk, bk2, Wv, bv2, mask)


def kernel(users_feature, items_feature, graph_src, graph_dst, graph_values,
           Wq, bq, Wk, bk, Wv, bv, mask):
    feats = jnp.concatenate([users_feature, items_feature], axis=0)
    table = feats.reshape(_N * 4, 16)
    src3 = graph_src.reshape(_GB, 1, _KB, 128)
    dst3 = graph_dst.reshape(_GB, 1, _KB, 128)
    packed = jnp.concatenate([src3, dst3], axis=1)
    vals = graph_values.reshape(_GB, _KB, 128)

    seg = _sc_propagate(table, packed, vals)

    users_rep = _combine(users_feature, lax.slice(seg, (0, 0), (_U, _D)), 4096)
    items_rep = _combine(items_feature, lax.slice(seg, (_U, 0), (_N, _D)), 4096)
    attn = _attention(items_feature, Wq, bq, Wk, bk, Wv, bv, mask)
    return users_rep, items_rep, attn


# speedup vs baseline: 18.5839x; 1.2456x over previous
import functools

import jax
import jax.numpy as jnp
from jax import lax
from jax.experimental import pallas as pl
from jax.experimental.pallas import tpu as pltpu
from jax.experimental.pallas import tpu_sc as plsc

_U, _I, _D = 65536, 4096, 64
_N = _U + _I
_E = 2097152
_KB = 4
_B = _KB * 128
_NTILES = 16
_ROWS_PER_TILE = _N // _NTILES
_ZR = _ROWS_PER_TILE // 16
_GB = _E // _B
_BLOCKS = _GB // _NTILES
_QUADS = _BLOCKS // 4


def _sc_propagate(table, packed, vals):
    mesh = plsc.VectorSubcoreMesh(core_axis_name="c", subcore_axis_name="s")

    @functools.partial(
        pl.kernel,
        mesh=mesh,
        compiler_params=pltpu.CompilerParams(use_tc_tiling_on_sc=False),
        out_type=jax.ShapeDtypeStruct((_N, _D), jnp.float32),
        scratch_types=[
            pltpu.VMEM((4, 2, _KB, 128), jnp.int32),
            pltpu.VMEM((4, _KB, 128), jnp.float32),
            pltpu.VMEM((4, _KB, 128), jnp.int32),
            pltpu.VMEM((4, _B, 16), jnp.float32),
            pltpu.VMEM((_ZR, 16), jnp.float32),
            pltpu.VMEM_SHARED((_N, 16), jnp.float32),
            [pltpu.SemaphoreType.DMA] * 4,
            [pltpu.SemaphoreType.DMA] * 4,
            [pltpu.SemaphoreType.DMA] * 4,
        ],
    )
    def prop(table_ref, pk_ref, val_ref, out_ref,
             pk_v, val_v, gidx_v, rows_v, zero_v, acc, semi, semg, sems):
        c = lax.axis_index("c")
        s = lax.axis_index("s")

        def zbody(r, carry):
            zero_v[r] = jnp.zeros((16,), jnp.float32)
            return carry
        lax.fori_loop(0, _ZR, zbody, 0)

        row_base = s * _ROWS_PER_TILE
        gblk_base = s * _BLOCKS

        def idx_load(b, u):
            pltpu.async_copy(pk_ref.at[gblk_base + b], pk_v.at[u], semi[u])
            pltpu.async_copy(val_ref.at[gblk_base + b], val_v.at[u], semi[u])

        def fire(b, u, k):
            pltpu.make_async_copy(pk_ref.at[0], pk_v.at[u], semi[u]).wait()
            pltpu.make_async_copy(val_ref.at[0], val_v.at[u], semi[u]).wait()
            for rr in range(_KB):
                for cc in range(8):
                    vsl = pk_v[u, 0, rr, pl.ds(cc * 16, 16)]
                    gidx_v[u, rr, pl.ds(cc * 16, 16)] = vsl * 4 + k
            for j in range(_KB):
                pltpu.async_copy(
                    table_ref.at[gidx_v.at[u, j]],
                    rows_v.at[u, pl.ds(j * 128, 128), :], semg[u])

        def scat_drain(u):
            pltpu.make_async_copy(
                table_ref.at[pl.ds(0, _B), :], rows_v.at[u], sems[u]).wait()

        def proc(u):
            pltpu.make_async_copy(
                table_ref.at[pl.ds(0, _B), :], rows_v.at[u], semg[u]).wait()
            for j in range(_KB):
                def sb(g, carry, j=j, u=u):
                    vv = val_v[u, j, pl.ds(g * 16, 16)]
                    base = j * 128 + g * 16
                    for t in range(16):
                        rows_v[u, base + t] = rows_v[u, base + t] * vv[t]
                    return carry
                lax.fori_loop(0, 8, sb, 0)
            for j in range(_KB):
                pltpu.async_copy(
                    rows_v.at[u, pl.ds(j * 128, 128), :],
                    acc.at[pk_v.at[u, 1, j]], sems[u], add=True)

        def pass_body(p, carry):
            k = c * 2 + p

            for z in range(16):
                pltpu.sync_copy(zero_v, acc.at[pl.ds(row_base + z * _ZR, _ZR), :])
            plsc.subcore_barrier()

            idx_load(0, 0)
            idx_load(1, 1)
            idx_load(2, 2)
            fire(0, 0, k)

            def quad_body(i, carry2, k=k):
                for q in range(4):
                    b = 4 * i + q
                    if q == 0:
                        @pl.when(i > 0)
                        def _():
                            scat_drain(3)
                    else:
                        scat_drain(q - 1)
                    if q == 0:
                        idx_load(b + 3, 3)
                    else:
                        @pl.when(i < _QUADS - 1)
                        def _(q=q, b=b):
                            idx_load(b + 3, (q + 3) % 4)
                    if q < 3:
                        fire(b + 1, q + 1, k)
                    else:
                        @pl.when(i < _QUADS - 1)
                        def _(b=b, k=k):
                            fire(b + 1, 0, k)
                    proc(q)
                return carry2
            lax.fori_loop(0, _QUADS, quad_body, 0)
            scat_drain(3)

            plsc.subcore_barrier()
            pltpu.sync_copy(
                acc.at[pl.ds(row_base, _ROWS_PER_TILE), :],
                out_ref.at[pl.ds(row_base, _ROWS_PER_TILE), pl.ds(k * 16, 16)])
            plsc.subcore_barrier()
            return carry

        lax.fori_loop(0, 2, pass_body, 0)

    return prop(table, packed, vals)


def _combine(feat, seg, br):
    n = feat.shape[0]

    def body(f_ref, g_ref, o_ref):
        f = f_ref[...]
        g = g_ref[...]
        ss = jnp.sum(g * g, axis=1, keepdims=True)
        denom = jnp.maximum(jnp.sqrt(ss), 2e-12)
        o_ref[...] = jnp.concatenate([f, g / denom], axis=1)

    return pl.pallas_call(
        body,
        grid=(n // br,),
        in_specs=[pl.BlockSpec((br, _D), lambda i: (i, 0)),
                  pl.BlockSpec((br, _D), lambda i: (i, 0))],
        out_specs=pl.BlockSpec((br, 2 * _D), lambda i: (i, 0)),
        out_shape=jax.ShapeDtypeStruct((n, 2 * _D), jnp.float32),
    )(feat, seg)


def _attention(items, Wq, bq, Wk, bk, Wv, bv, mask, bq_blk=512):
    bq2 = bq.reshape(1, _D)
    bk2 = bk.reshape(1, _D)
    bv2 = bv.reshape(1, _D)

    def body(q_ref, kv_ref, wq_ref, bq_ref, wk_ref, bk_ref, wv_ref, bv_ref,
             m_ref, o_ref):
        f32 = jnp.float32
        q = jnp.dot(q_ref[...], wq_ref[...], preferred_element_type=f32) + bq_ref[...]
        kv = kv_ref[...]
        kk = jnp.dot(kv, wk_ref[...], preferred_element_type=f32) + bk_ref[...]
        vv = jnp.dot(kv, wv_ref[...], preferred_element_type=f32) + bv_ref[...]
        s = lax.dot_general(q, kk, (((1,), (1,)), ((), ())),
                            preferred_element_type=f32)
        s = s * m_ref[...] * (1.0 / 8.0)
        m = jnp.max(s, axis=1, keepdims=True)
        p = jnp.exp(s - m)
        denom = jnp.sum(p, axis=1, keepdims=True)
        o = lax.dot_general(p, vv, (((1,), (0,)), ((), ())),
                            preferred_element_type=f32)
        o_ref[...] = o / denom

    return pl.pallas_call(
        body,
        grid=(_I // bq_blk,),
        in_specs=[
            pl.BlockSpec((bq_blk, _D), lambda i: (i, 0)),
            pl.BlockSpec((_I, _D), lambda i: (0, 0)),
            pl.BlockSpec((_D, _D), lambda i: (0, 0)),
            pl.BlockSpec((1, _D), lambda i: (0, 0)),
            pl.BlockSpec((_D, _D), lambda i: (0, 0)),
            pl.BlockSpec((1, _D), lambda i: (0, 0)),
            pl.BlockSpec((_D, _D), lambda i: (0, 0)),
            pl.BlockSpec((1, _D), lambda i: (0, 0)),
            pl.BlockSpec((bq_blk, _I), lambda i: (i, 0)),
        ],
        out_specs=pl.BlockSpec((bq_blk, _D), lambda i: (i, 0)),
        out_shape=jax.ShapeDtypeStruct((_I, _D), jnp.float32),
    )(items, items, Wq, bq2, Wk, bk2, Wv, bv2, mask)


def kernel(users_feature, items_feature, graph_src, graph_dst, graph_values,
           Wq, bq, Wk, bk, Wv, bv, mask):
    feats = jnp.concatenate([users_feature, items_feature], axis=0)
    table = feats.reshape(_N * 4, 16)
    src3 = graph_src.reshape(_GB, 1, _KB, 128)
    dst3 = graph_dst.reshape(_GB, 1, _KB, 128)
    packed = jnp.concatenate([src3, dst3], axis=1)
    vals = graph_values.reshape(_GB, _KB, 128)

    seg = _sc_propagate(table, packed, vals)

    users_rep = _combine(users_feature, lax.slice(seg, (0, 0), (_U, _D)), 4096)
    items_rep = _combine(items_feature, lax.slice(seg, (_U, 0), (_N, _D)), 4096)
    attn = _attention(items_feature, Wq, bq, Wk, bk, Wv, bv, mask)
    return users_rep, items_rep, attn


# speedup vs baseline: 21.6615x; 1.1656x over previous
import functools

import jax
import jax.numpy as jnp
from jax import lax
from jax.experimental import pallas as pl
from jax.experimental.pallas import tpu as pltpu
from jax.experimental.pallas import tpu_sc as plsc

_U, _I, _D = 65536, 4096, 64
_N = _U + _I
_E = 2097152
_KB = 4
_B = _KB * 128
_NTILES = 16
_ROWS_PER_TILE = _N // _NTILES
_ZR = _ROWS_PER_TILE // 16
_GB = _E // _B
_BLOCKS = _GB // _NTILES
_QUADS = _BLOCKS // 4


def _sc_propagate(table, srcs, dsts, vals):
    mesh = plsc.VectorSubcoreMesh(core_axis_name="c", subcore_axis_name="s")

    @functools.partial(
        pl.kernel,
        mesh=mesh,
        compiler_params=pltpu.CompilerParams(use_tc_tiling_on_sc=False),
        out_type=jax.ShapeDtypeStruct((_N, _D), jnp.float32),
        scratch_types=[
            pltpu.VMEM((4, _KB, 128), jnp.int32),
            pltpu.VMEM((4, _KB, 128), jnp.int32),
            pltpu.VMEM((4, _KB, 128), jnp.float32),
            pltpu.VMEM((4, _KB, 128), jnp.int32),
            pltpu.VMEM((4, _B, 16), jnp.float32),
            pltpu.VMEM((_ZR, 16), jnp.float32),
            pltpu.VMEM_SHARED((_N, 16), jnp.float32),
            [pltpu.SemaphoreType.DMA] * 4,
            [pltpu.SemaphoreType.DMA] * 4,
            [pltpu.SemaphoreType.DMA] * 4,
        ],
    )
    def prop(table_ref, src_ref, dst_ref, val_ref, out_ref,
             src_v, dst_v, val_v, gidx_v, rows_v, zero_v, acc, semi, semg, sems):
        c = lax.axis_index("c")
        s = lax.axis_index("s")

        def zbody(r, carry):
            zero_v[r] = jnp.zeros((16,), jnp.float32)
            return carry
        lax.fori_loop(0, _ZR, zbody, 0)

        row_base = s * _ROWS_PER_TILE
        gblk_base = s * _BLOCKS

        def idx_load(b, u):
            pltpu.async_copy(src_ref.at[gblk_base + b], src_v.at[u], semi[u])
            pltpu.async_copy(dst_ref.at[gblk_base + b], dst_v.at[u], semi[u])
            pltpu.async_copy(val_ref.at[gblk_base + b], val_v.at[u], semi[u])

        def fire(b, u, k):
            pltpu.make_async_copy(src_ref.at[0], src_v.at[u], semi[u]).wait()
            pltpu.make_async_copy(dst_ref.at[0], dst_v.at[u], semi[u]).wait()
            pltpu.make_async_copy(val_ref.at[0], val_v.at[u], semi[u]).wait()
            for rr in range(_KB):
                for cc in range(8):
                    vsl = src_v[u, rr, pl.ds(cc * 16, 16)]
                    gidx_v[u, rr, pl.ds(cc * 16, 16)] = vsl * 4 + k
            for j in range(_KB):
                pltpu.async_copy(
                    table_ref.at[gidx_v.at[u, j]],
                    rows_v.at[u, pl.ds(j * 128, 128), :], semg[u])

        def scat_drain(u):
            pltpu.make_async_copy(
                table_ref.at[pl.ds(0, _B), :], rows_v.at[u], sems[u]).wait()

        def proc(u):
            pltpu.make_async_copy(
                table_ref.at[pl.ds(0, _B), :], rows_v.at[u], semg[u]).wait()
            for j in range(_KB):
                def sb(g, carry, j=j, u=u):
                    vv = val_v[u, j, pl.ds(g * 16, 16)]
                    base = j * 128 + g * 16
                    for t in range(16):
                        rows_v[u, base + t] = rows_v[u, base + t] * vv[t]
                    return carry
                lax.fori_loop(0, 8, sb, 0)
            for j in range(_KB):
                pltpu.async_copy(
                    rows_v.at[u, pl.ds(j * 128, 128), :],
                    acc.at[dst_v.at[u, j]], sems[u], add=True)

        def pass_body(p, carry):
            k = c * 2 + p

            for z in range(16):
                pltpu.sync_copy(zero_v, acc.at[pl.ds(row_base + z * _ZR, _ZR), :])
            plsc.subcore_barrier()

            idx_load(0, 0)
            idx_load(1, 1)
            fire(0, 0, k)

            def quad_body(i, carry2, k=k):
                for q in range(4):
                    b = 4 * i + q
                    if q >= 2:
                        scat_drain(q - 2)
                    else:
                        @pl.when(i > 0)
                        def _(q=q):
                            scat_drain((q + 2) % 4)
                    if q < 2:
                        idx_load(b + 2, (q + 2) % 4)
                    else:
                        @pl.when(i < _QUADS - 1)
                        def _(q=q, b=b):
                            idx_load(b + 2, (q + 2) % 4)
                    if q < 3:
                        fire(b + 1, q + 1, k)
                    else:
                        @pl.when(i < _QUADS - 1)
                        def _(b=b, k=k):
                            fire(b + 1, 0, k)
                    proc(q)
                return carry2
            lax.fori_loop(0, _QUADS, quad_body, 0)
            scat_drain(2)
            scat_drain(3)

            plsc.subcore_barrier()
            pltpu.sync_copy(
                acc.at[pl.ds(row_base, _ROWS_PER_TILE), :],
                out_ref.at[pl.ds(row_base, _ROWS_PER_TILE), pl.ds(k * 16, 16)])
            plsc.subcore_barrier()
            return carry

        lax.fori_loop(0, 2, pass_body, 0)

    return prop(table, srcs, dsts, vals)


def _combine(feat, seg, br, seg_row_off):
    n = feat.shape[0]

    def body(f_ref, g_ref, o_ref):
        f = f_ref[...]
        g = g_ref[...]
        ss = jnp.sum(g * g, axis=1, keepdims=True)
        denom = jnp.maximum(jnp.sqrt(ss), 2e-12)
        o_ref[...] = jnp.concatenate([f, g / denom], axis=1)

    return pl.pallas_call(
        body,
        grid=(n // br,),
        in_specs=[pl.BlockSpec((br, _D), lambda i: (i, 0)),
                  pl.BlockSpec((br, _D), lambda i, o=seg_row_off: (i + o, 0))],
        out_specs=pl.BlockSpec((br, 2 * _D), lambda i: (i, 0)),
        out_shape=jax.ShapeDtypeStruct((n, 2 * _D), jnp.float32),
    )(feat, seg)


def _attention(items, Wq, bq, Wk, bk, Wv, bv, mask, bq_blk=512):
    bq2 = bq.reshape(1, _D)
    bk2 = bk.reshape(1, _D)
    bv2 = bv.reshape(1, _D)

    def body(q_ref, kv_ref, wq_ref, bq_ref, wk_ref, bk_ref, wv_ref, bv_ref,
             m_ref, o_ref):
        f32 = jnp.float32
        q = jnp.dot(q_ref[...], wq_ref[...], preferred_element_type=f32) + bq_ref[...]
        kv = kv_ref[...]
        kk = jnp.dot(kv, wk_ref[...], preferred_element_type=f32) + bk_ref[...]
        vv = jnp.dot(kv, wv_ref[...], preferred_element_type=f32) + bv_ref[...]
        s = lax.dot_general(q, kk, (((1,), (1,)), ((), ())),
                            preferred_element_type=f32)
        s = s * m_ref[...] * (1.0 / 8.0)
        m = jnp.max(s, axis=1, keepdims=True)
        p = jnp.exp(s - m)
        denom = jnp.sum(p, axis=1, keepdims=True)
        o = lax.dot_general(p, vv, (((1,), (0,)), ((), ())),
                            preferred_element_type=f32)
        o_ref[...] = o / denom

    return pl.pallas_call(
        body,
        grid=(_I // bq_blk,),
        in_specs=[
            pl.BlockSpec((bq_blk, _D), lambda i: (i, 0)),
            pl.BlockSpec((_I, _D), lambda i: (0, 0)),
            pl.BlockSpec((_D, _D), lambda i: (0, 0)),
            pl.BlockSpec((1, _D), lambda i: (0, 0)),
            pl.BlockSpec((_D, _D), lambda i: (0, 0)),
            pl.BlockSpec((1, _D), lambda i: (0, 0)),
            pl.BlockSpec((_D, _D), lambda i: (0, 0)),
            pl.BlockSpec((1, _D), lambda i: (0, 0)),
            pl.BlockSpec((bq_blk, _I), lambda i: (i, 0)),
        ],
        out_specs=pl.BlockSpec((bq_blk, _D), lambda i: (i, 0)),
        out_shape=jax.ShapeDtypeStruct((_I, _D), jnp.float32),
    )(items, items, Wq, bq2, Wk, bk2, Wv, bv2, mask)


def kernel(users_feature, items_feature, graph_src, graph_dst, graph_values,
           Wq, bq, Wk, bk, Wv, bv, mask):
    feats = jnp.concatenate([users_feature, items_feature], axis=0)
    table = feats.reshape(_N * 4, 16)
    srcs = graph_src.reshape(_GB, _KB, 128)
    dsts = graph_dst.reshape(_GB, _KB, 128)
    vals = graph_values.reshape(_GB, _KB, 128)

    seg = _sc_propagate(table, srcs, dsts, vals)

    users_rep = _combine(users_feature, seg, 4096, 0)
    items_rep = _combine(items_feature, seg, 4096, _U // 4096)
    attn = _attention(items_feature, Wq, bq, Wk, bk, Wv, bv, mask)
    return users_rep, items_rep, attn
